# baseline (speedup 1.0000x reference)
import functools

import jax
import jax.numpy as jnp
from jax import lax
from jax.experimental import pallas as pl
from jax.experimental.pallas import tpu as pltpu
from jax.experimental.pallas import tpu_sc as plsc

V = 1000000
D = 32
B = 16384 * 26
NC = 2
NS = 16
NW = NC * NS
B_PER_W = B // NW
N_CHUNK = 8
C = B_PER_W // N_CHUNK

RK = 1024
N_GCHUNK = -(-V // RK)
N_RCHUNK = -(-N_GCHUNK // NW)

_mesh = plsc.VectorSubcoreMesh(core_axis_name="c", subcore_axis_name="s")
_params = pltpu.CompilerParams(use_tc_tiling_on_sc=False)
_params_nl = pltpu.CompilerParams(use_tc_tiling_on_sc=False, needs_layout_passes=False)


@functools.partial(
    pl.kernel,
    mesh=_mesh,
    out_type=jax.ShapeDtypeStruct((V, D), jnp.float32),
    compiler_params=_params_nl,
    scratch_types=[
        pltpu.VMEM((D, RK), jnp.float32),
        pltpu.VMEM((RK, D), jnp.float32),
    ],
)
def _repack(tt_hbm, out_hbm, tin, tout):
    wid = lax.axis_index("s") * NC + lax.axis_index("c")
    iota16 = lax.iota(jnp.int32, 16)

    def chunk(cc, carry):
        r0 = jnp.minimum((wid + cc * NW) * RK, V - RK)
        pltpu.sync_copy(tt_hbm.at[:, pl.ds(r0, RK)], tin)

        def j_body(j, c2):
            ridx = j * 16 + iota16
            for d in range(D):
                v = tin[d, pl.ds(j * 16, 16)]
                plsc.store_scatter(
                    tout, [ridx, jnp.full((16,), d, jnp.int32)], v)
            return c2

        lax.fori_loop(0, RK // 16, j_body, 0)
        pltpu.sync_copy(tout, out_hbm.at[pl.ds(r0, RK)])
        return carry

    lax.fori_loop(0, N_RCHUNK, chunk, 0)


@functools.partial(
    pl.kernel,
    mesh=_mesh,
    out_type=jax.ShapeDtypeStruct((B, D), jnp.float32),
    compiler_params=_params,
    scratch_types=[
        pltpu.VMEM((B_PER_W,), jnp.int32),
        pltpu.VMEM((C, D), jnp.float32),
        pltpu.VMEM((C, D), jnp.float32),
        pltpu.SemaphoreType.DMA,
        pltpu.SemaphoreType.DMA,
        pltpu.SemaphoreType.DMA,
        pltpu.SemaphoreType.DMA,
    ],
)
def _emb_gather(x_hbm, table_hbm, out_hbm, idx_v, rows0, rows1,
                gsem0, gsem1, wsem0, wsem1):
    wid = lax.axis_index("s") * NC + lax.axis_index("c")
    base = wid * B_PER_W

    pltpu.sync_copy(x_hbm.at[pl.ds(base, B_PER_W)], idx_v)

    bufs = (rows0, rows1)
    gsems = (gsem0, gsem1)
    wsems = (wsem0, wsem1)

    gcp = [None] * N_CHUNK
    wcp = [None] * N_CHUNK
    for i in range(N_CHUNK):
        b = i % 2
        if i >= 2:
            wcp[i - 2].wait()
        gcp[i] = pltpu.async_copy(
            table_hbm.at[idx_v.at[pl.ds(i * C, C)]], bufs[b], gsems[b])
        if i >= 1:
            pb = (i - 1) % 2
            gcp[i - 1].wait()
            wcp[i - 1] = pltpu.async_copy(
                bufs[pb], out_hbm.at[pl.ds(base + (i - 1) * C, C)], wsems[pb])

    last = N_CHUNK - 1
    lb = last % 2
    gcp[last].wait()
    wcp[last] = pltpu.async_copy(
        bufs[lb], out_hbm.at[pl.ds(base + last * C, C)], wsems[lb])
    wcp[last - 1].wait()
    wcp[last].wait()


def kernel(x, embedding_tables):
    xf = jax.lax.optimization_barrier(x.astype(jnp.int32).reshape(B // 128, 128))
    x_flat = xf.reshape(B)
    table_rm = _repack(embedding_tables.T)
    out = _emb_gather(x_flat, table_rm)
    out128 = jax.lax.optimization_barrier(out.reshape(B * D // 128, 128))
    return out128.reshape(16384, 26, D)


# speedup vs baseline: 1.0293x; 1.0293x over previous
import functools

import jax
import jax.numpy as jnp
from jax import lax
from jax.experimental import pallas as pl
from jax.experimental.pallas import tpu as pltpu
from jax.experimental.pallas import tpu_sc as plsc

V = 1000000
D = 32
B = 16384 * 26
NC = 2
NS = 16
NW = NC * NS
B_PER_W = B // NW
N_CHUNK = 8
C = B_PER_W // N_CHUNK

RK = 1024
N_GCHUNK = -(-V // RK)
N_RCHUNK = -(-N_GCHUNK // NW)

_mesh = plsc.VectorSubcoreMesh(core_axis_name="c", subcore_axis_name="s")
_params = pltpu.CompilerParams(use_tc_tiling_on_sc=False)
_params_nl = pltpu.CompilerParams(use_tc_tiling_on_sc=False, needs_layout_passes=False)


@functools.partial(
    pl.kernel,
    mesh=_mesh,
    out_type=jax.ShapeDtypeStruct((V * D,), jnp.float32),
    compiler_params=_params_nl,
    scratch_types=[
        pltpu.VMEM((D, RK), jnp.float32),
        pltpu.VMEM((RK * D,), jnp.float32),
    ],
)
def _repack(tt_hbm, out_hbm, tin, tflat):
    wid = lax.axis_index("s") * NC + lax.axis_index("c")
    iota16 = lax.iota(jnp.int32, 16)

    def chunk(cc, carry):
        r0 = jnp.minimum((wid + cc * NW) * RK, V - RK)
        pltpu.sync_copy(tt_hbm.at[:, pl.ds(r0, RK)], tin)

        @plsc.parallel_loop(0, RK // 16, unroll=4)
        def j_body(j):
            base32 = (j * 16 + iota16) * D
            for d in range(D):
                v = tin[d, pl.ds(j * 16, 16)]
                plsc.store_scatter(tflat, [base32 + d], v)

        pltpu.sync_copy(tflat, out_hbm.at[pl.ds(r0 * D, RK * D)])
        return carry

    lax.fori_loop(0, N_RCHUNK, chunk, 0)


@functools.partial(
    pl.kernel,
    mesh=_mesh,
    out_type=jax.ShapeDtypeStruct((B, D), jnp.float32),
    compiler_params=_params,
    scratch_types=[
        pltpu.VMEM((B_PER_W,), jnp.int32),
        pltpu.VMEM((C, D), jnp.float32),
        pltpu.VMEM((C, D), jnp.float32),
        pltpu.SemaphoreType.DMA,
        pltpu.SemaphoreType.DMA,
        pltpu.SemaphoreType.DMA,
        pltpu.SemaphoreType.DMA,
    ],
)
def _emb_gather(x_hbm, table_hbm, out_hbm, idx_v, rows0, rows1,
                gsem0, gsem1, wsem0, wsem1):
    wid = lax.axis_index("s") * NC + lax.axis_index("c")
    base = wid * B_PER_W

    pltpu.sync_copy(x_hbm.at[pl.ds(base, B_PER_W)], idx_v)

    bufs = (rows0, rows1)
    gsems = (gsem0, gsem1)
    wsems = (wsem0, wsem1)

    gcp = [None] * N_CHUNK
    wcp = [None] * N_CHUNK
    for i in range(N_CHUNK):
        b = i % 2
        if i >= 2:
            wcp[i - 2].wait()
        gcp[i] = pltpu.async_copy(
            table_hbm.at[idx_v.at[pl.ds(i * C, C)]], bufs[b], gsems[b])
        if i >= 1:
            pb = (i - 1) % 2
            gcp[i - 1].wait()
            wcp[i - 1] = pltpu.async_copy(
                bufs[pb], out_hbm.at[pl.ds(base + (i - 1) * C, C)], wsems[pb])

    last = N_CHUNK - 1
    lb = last % 2
    gcp[last].wait()
    wcp[last] = pltpu.async_copy(
        bufs[lb], out_hbm.at[pl.ds(base + last * C, C)], wsems[lb])
    wcp[last - 1].wait()
    wcp[last].wait()


def kernel(x, embedding_tables):
    xf = jax.lax.optimization_barrier(x.astype(jnp.int32).reshape(B // 128, 128))
    x_flat = xf.reshape(B)
    table_rm = _repack(embedding_tables.T).reshape(V, D)
    out = _emb_gather(x_flat, table_rm)
    out128 = jax.lax.optimization_barrier(out.reshape(B * D // 128, 128))
    return out128.reshape(16384, 26, D)


# speedup vs baseline: 4.5067x; 4.3784x over previous
import functools

import jax
import jax.numpy as jnp
from jax import lax
from jax.experimental import pallas as pl
from jax.experimental.pallas import tpu as pltpu
from jax.experimental.pallas import tpu_sc as plsc

D = 32
B = 16384 * 26
NC = 2
NS = 16
NW = NC * NS
B_PER_W = B // NW
N_CHUNK = 8
C = B_PER_W // N_CHUNK

_mesh = plsc.VectorSubcoreMesh(core_axis_name="c", subcore_axis_name="s")


@functools.partial(
    pl.kernel,
    mesh=_mesh,
    out_type=jax.ShapeDtypeStruct((B, D), jnp.float32),
    compiler_params=pltpu.CompilerParams(use_tc_tiling_on_sc=False),
    scratch_types=[
        pltpu.VMEM((B_PER_W,), jnp.int32),
        pltpu.VMEM((C, D), jnp.float32),
        pltpu.VMEM((C, D), jnp.float32),
        pltpu.SemaphoreType.DMA,
        pltpu.SemaphoreType.DMA,
        pltpu.SemaphoreType.DMA,
        pltpu.SemaphoreType.DMA,
    ],
)
def _emb_gather(x_hbm, table_hbm, out_hbm, idx_v, rows0, rows1,
                gsem0, gsem1, wsem0, wsem1):
    wid = lax.axis_index("s") * NC + lax.axis_index("c")
    base = wid * B_PER_W

    pltpu.sync_copy(x_hbm.at[pl.ds(base, B_PER_W)], idx_v)

    bufs = (rows0, rows1)
    gsems = (gsem0, gsem1)
    wsems = (wsem0, wsem1)

    gcp = [None] * N_CHUNK
    wcp = [None] * N_CHUNK
    for i in range(N_CHUNK):
        b = i % 2
        if i >= 2:
            wcp[i - 2].wait()
        gcp[i] = pltpu.async_copy(
            table_hbm.at[idx_v.at[pl.ds(i * C, C)]], bufs[b], gsems[b])
        if i >= 1:
            pb = (i - 1) % 2
            gcp[i - 1].wait()
            wcp[i - 1] = pltpu.async_copy(
                bufs[pb], out_hbm.at[pl.ds(base + (i - 1) * C, C)], wsems[pb])

    last = N_CHUNK - 1
    lb = last % 2
    gcp[last].wait()
    wcp[last] = pltpu.async_copy(
        bufs[lb], out_hbm.at[pl.ds(base + last * C, C)], wsems[lb])
    wcp[last - 1].wait()
    wcp[last].wait()


def kernel(x, embedding_tables):
    x_flat = x.reshape(-1).astype(jnp.int32)
    out = _emb_gather(x_flat, embedding_tables)
    return out.reshape(x.shape + (D,))
